# baseline (speedup 1.0000x reference)
import dataclasses
import functools

import jax
import jax.numpy as jnp
from jax import lax
from jax.experimental import pallas as pl
from jax.experimental.pallas import tpu as pltpu
from jax.experimental.pallas import tpu_sc as plsc

DIM = 64
BATCH = 16384
ROW_W = 128
NUM_REL = 100000
FOLD = 65536
NUM_CORES = 2
NUM_SUBCORES = 16
NUM_WORKERS = NUM_CORES * NUM_SUBCORES
ROWS_PER_WORKER = BATCH // NUM_WORKERS
STAGE_ROWS = 128
NUM_STAGES = ROWS_PER_WORKER // STAGE_ROWS
LANES = 16
ROW_CHUNKS = ROW_W // LANES
GROUPS_PER_STAGE = STAGE_ROWS // LANES

TR_COLS = 16384


def _pack_diag_kernel(lo_ref, hi_ref, out_ref):
  i = pl.program_id(0)
  a = lo_ref[...].T
  b = hi_ref[...].T
  rid = (FOLD + i * TR_COLS
         + jax.lax.broadcasted_iota(jnp.int32, (TR_COLS, DIM), 0))
  b = jnp.where(rid < NUM_REL, b, 0.0)
  out_ref[...] = jnp.concatenate([a, b], axis=1)


def _pack_diag(dt):
  return pl.pallas_call(
      _pack_diag_kernel,
      grid=(FOLD // TR_COLS,),
      in_specs=[
          pl.BlockSpec((DIM, TR_COLS), lambda i: (0, i)),
          pl.BlockSpec(
              (DIM, TR_COLS),
              lambda i: (0, jnp.minimum(i + FOLD // TR_COLS,
                                        (NUM_REL - 1) // TR_COLS)),
          ),
      ],
      out_specs=pl.BlockSpec((TR_COLS, ROW_W), lambda i: (i, 0)),
      out_shape=jax.ShapeDtypeStruct((FOLD, ROW_W), jnp.float32),
  )(dt, dt)


QC_COLS = 4096


def _pack_q_kernel(h_ref, t_ref, p_ref, out_ref):
  q = h_ref[...] * t_ref[...]
  lo = p_ref[...] == 0
  out_ref[...] = jnp.concatenate(
      [jnp.where(lo, q, 0.0).T, jnp.where(lo, 0.0, q).T], axis=1)


def _pack_q(ht, tt, parity):
  return pl.pallas_call(
      _pack_q_kernel,
      grid=(BATCH // QC_COLS,),
      in_specs=[
          pl.BlockSpec((DIM, QC_COLS), lambda i: (0, i)),
          pl.BlockSpec((DIM, QC_COLS), lambda i: (0, i)),
          pl.BlockSpec((1, QC_COLS), lambda i: (0, i)),
      ],
      out_specs=pl.BlockSpec((QC_COLS, ROW_W), lambda i: (i, 0)),
      out_shape=jax.ShapeDtypeStruct((BATCH, ROW_W), jnp.float32),
  )(ht, tt, parity)


def _sc_kernel(diagp_hbm, idx_hbm, q_hbm, out_hbm,
               idx_v, rel0, rel1, rel2, rel3, qb0, qb1, out_v,
               sem_q0, sem_q1, sem_g0, sem_g1, sem_g2, sem_g3):
  wid = lax.axis_index("s") * NUM_CORES + lax.axis_index("c")
  base = wid * ROWS_PER_WORKER

  pltpu.sync_copy(idx_hbm.at[wid], idx_v)

  rel = (rel0, rel1, rel2, rel3)
  sems = (sem_g0, sem_g1, sem_g2, sem_g3)
  qbufs = (qb0, qb1)
  qsems = (sem_q0, sem_q1)
  lane = lax.iota(jnp.int32, LANES)

  def qcopy(s):
    return pltpu.async_copy(
        q_hbm.at[pl.ds(base + s * STAGE_ROWS, STAGE_ROWS)],
        qbufs[s % 2], qsems[s % 2])

  def compute_stage(s, relbuf, qbuf):
    @pl.loop(0, GROUPS_PER_STAGE)
    def _(g):
      res = [jnp.zeros((LANES,), jnp.float32) for _ in range(2)]
      for k in range(LANES):
        li = g * LANES + k
        acc = None
        for c in range(ROW_CHUNKS):
          term = (qbuf[li, pl.ds(c * LANES, LANES)]
                  * relbuf[li, pl.ds(c * LANES, LANES)])
          acc = term if acc is None else acc + term
        ch = k & 1
        res[ch] = jnp.where(lane == k, jnp.sum(acc), res[ch])
      out_v[pl.ds(s * STAGE_ROWS + g * LANES, LANES)] = res[0] + res[1]

  qpend = [qcopy(0), qcopy(1)]
  gathers = [
      pltpu.async_copy(diagp_hbm.at[idx_v.at[s]], rel[s], sems[s])
      for s in range(NUM_STAGES)
  ]
  for s in range(NUM_STAGES):
    qpend[s % 2].wait()
    gathers[s].wait()
    compute_stage(s, rel[s], qbufs[s % 2])
    if s + 2 < NUM_STAGES:
      qpend[s % 2] = qcopy(s + 2)

  pltpu.sync_copy(out_v, out_hbm.at[pl.ds(base, ROWS_PER_WORKER)])


@jax.jit
def _dist_mult(h, r, t, diag):
  r32 = r.astype(jnp.int32)
  idx = (r32 & (FOLD - 1)).reshape(NUM_WORKERS, NUM_STAGES, STAGE_ROWS)
  parity = (r32 >> 16).reshape(1, BATCH)
  diagp = _pack_diag(diag.T)
  q = _pack_q(h.T, t.T, parity)
  mesh = plsc.VectorSubcoreMesh(core_axis_name="c", subcore_axis_name="s")
  cp = pltpu.CompilerParams()
  for field, value in (("needs_layout_passes", False),
                       ("use_tc_tiling_on_sc", True)):
    if field in pltpu.CompilerParams.__dataclass_fields__:
      cp = dataclasses.replace(cp, **{field: value})
  run = pl.kernel(
      _sc_kernel,
      out_type=jax.ShapeDtypeStruct((BATCH,), jnp.float32),
      mesh=mesh,
      compiler_params=cp,
      scratch_types=[
          pltpu.VMEM((NUM_STAGES, STAGE_ROWS), jnp.int32),
          pltpu.VMEM((STAGE_ROWS, ROW_W), jnp.float32),
          pltpu.VMEM((STAGE_ROWS, ROW_W), jnp.float32),
          pltpu.VMEM((STAGE_ROWS, ROW_W), jnp.float32),
          pltpu.VMEM((STAGE_ROWS, ROW_W), jnp.float32),
          pltpu.VMEM((STAGE_ROWS, ROW_W), jnp.float32),
          pltpu.VMEM((STAGE_ROWS, ROW_W), jnp.float32),
          pltpu.VMEM((ROWS_PER_WORKER,), jnp.float32),
          pltpu.SemaphoreType.DMA,
          pltpu.SemaphoreType.DMA,
          pltpu.SemaphoreType.DMA,
          pltpu.SemaphoreType.DMA,
          pltpu.SemaphoreType.DMA,
          pltpu.SemaphoreType.DMA,
      ],
  )
  return run(diagp, idx, q)


def kernel(h, r, t, diag):
  return _dist_mult(h, r, t, diag)


# speedup vs baseline: 1.3822x; 1.3822x over previous
import dataclasses
import functools

import jax
import jax.numpy as jnp
from jax import lax
from jax.experimental import pallas as pl
from jax.experimental.pallas import tpu as pltpu
from jax.experimental.pallas import tpu_sc as plsc

DIM = 64
BATCH = 16384
PAD_DIM = 128
NUM_REL = 100000
NUM_CORES = 2
NUM_SUBCORES = 16
NUM_WORKERS = NUM_CORES * NUM_SUBCORES
ROWS_PER_WORKER = BATCH // NUM_WORKERS
STAGE_ROWS = 128
NUM_STAGES = ROWS_PER_WORKER // STAGE_ROWS
LANES = 16
DIM_CHUNKS = DIM // LANES
GROUPS_PER_STAGE = STAGE_ROWS // LANES

TR_COLS = 16384


def _pack_diag_kernel(dt_ref, out_ref):
  x = dt_ref[...].T
  out_ref[...] = jnp.concatenate(
      [x, jnp.zeros((TR_COLS, PAD_DIM - DIM), jnp.float32)], axis=1)


def _pack_diag(dt):
  return pl.pallas_call(
      _pack_diag_kernel,
      grid=(-(-NUM_REL // TR_COLS),),
      in_specs=[pl.BlockSpec((DIM, TR_COLS), lambda i: (0, i))],
      out_specs=pl.BlockSpec((TR_COLS, PAD_DIM), lambda i: (i, 0)),
      out_shape=jax.ShapeDtypeStruct((NUM_REL, PAD_DIM), jnp.float32),
  )(dt)


QC_COLS = 4096
Q_ROWS = BATCH // 2


def _pack_q_kernel(ha_ref, ta_ref, hb_ref, tb_ref, out_ref):
  a = ha_ref[...] * ta_ref[...]
  b = hb_ref[...] * tb_ref[...]
  out_ref[...] = jnp.concatenate([a.T, b.T], axis=1)


def _pack_q(ht, tt):
  return pl.pallas_call(
      _pack_q_kernel,
      grid=(Q_ROWS // QC_COLS,),
      in_specs=[
          pl.BlockSpec((DIM, QC_COLS), lambda i: (0, i)),
          pl.BlockSpec((DIM, QC_COLS), lambda i: (0, i)),
          pl.BlockSpec((DIM, QC_COLS), lambda i: (0, i + Q_ROWS // QC_COLS)),
          pl.BlockSpec((DIM, QC_COLS), lambda i: (0, i + Q_ROWS // QC_COLS)),
      ],
      out_specs=pl.BlockSpec((QC_COLS, PAD_DIM), lambda i: (i, 0)),
      out_shape=jax.ShapeDtypeStruct((Q_ROWS, PAD_DIM), jnp.float32),
  )(ht, tt, ht, tt)


def _sc_kernel(diagp_hbm, idx_hbm, q_hbm, out_hbm,
               idx_v, rel0, rel1, rel2, rel3, q_v, out_v,
               sem_q, sem_g0, sem_g1, sem_g2, sem_g3):
  wid = lax.axis_index("s") * NUM_CORES + lax.axis_index("c")
  half = ROWS_PER_WORKER // 2
  base = wid * half

  pltpu.sync_copy(idx_hbm.at[wid], idx_v)
  copy_q = pltpu.async_copy(q_hbm.at[pl.ds(base, half)], q_v, sem_q)

  rel = (rel0, rel1, rel2, rel3)
  sems = (sem_g0, sem_g1, sem_g2, sem_g3)
  lane = lax.iota(jnp.int32, LANES)

  def compute_stage(s, relbuf):
    qoff = (s // 2) * DIM

    @pl.loop(0, GROUPS_PER_STAGE)
    def _(g):
      res = [jnp.zeros((LANES,), jnp.float32) for _ in range(2)]
      for k in range(LANES):
        li = g * LANES + k
        qrow = (s % 2) * STAGE_ROWS + g * LANES + k
        acc = None
        for c in range(DIM_CHUNKS):
          term = (q_v[qrow, pl.ds(qoff + c * LANES, LANES)]
                  * relbuf[li, pl.ds(c * LANES, LANES)])
          acc = term if acc is None else acc + term
        ch = k & 1
        res[ch] = jnp.where(lane == k, jnp.sum(acc), res[ch])
      out_v[pl.ds(s * STAGE_ROWS + g * LANES, LANES)] = res[0] + res[1]

  gathers = [
      pltpu.async_copy(diagp_hbm.at[idx_v.at[s]], rel[s], sems[s])
      for s in range(NUM_STAGES)
  ]
  copy_q.wait()
  for s in range(NUM_STAGES):
    gathers[s].wait()
    compute_stage(s, rel[s])

  pltpu.sync_copy(out_v.at[pl.ds(0, half)], out_hbm.at[pl.ds(base, half)])
  pltpu.sync_copy(out_v.at[pl.ds(half, half)],
                  out_hbm.at[pl.ds(Q_ROWS + base, half)])


@jax.jit
def _dist_mult(h, r, t, diag):
  idx = (r.astype(jnp.int32)
         .reshape(2, NUM_WORKERS, ROWS_PER_WORKER // 2)
         .transpose(1, 0, 2)
         .reshape(NUM_WORKERS, NUM_STAGES, STAGE_ROWS))
  diagp = _pack_diag(diag.T)
  q = _pack_q(h.T, t.T)
  mesh = plsc.VectorSubcoreMesh(core_axis_name="c", subcore_axis_name="s")
  cp = pltpu.CompilerParams()
  for field, value in (("needs_layout_passes", False),
                       ("use_tc_tiling_on_sc", True)):
    if field in pltpu.CompilerParams.__dataclass_fields__:
      cp = dataclasses.replace(cp, **{field: value})
  run = pl.kernel(
      _sc_kernel,
      out_type=jax.ShapeDtypeStruct((BATCH,), jnp.float32),
      mesh=mesh,
      compiler_params=cp,
      scratch_types=[
          pltpu.VMEM((NUM_STAGES, STAGE_ROWS), jnp.int32),
          pltpu.VMEM((STAGE_ROWS, PAD_DIM), jnp.float32),
          pltpu.VMEM((STAGE_ROWS, PAD_DIM), jnp.float32),
          pltpu.VMEM((STAGE_ROWS, PAD_DIM), jnp.float32),
          pltpu.VMEM((STAGE_ROWS, PAD_DIM), jnp.float32),
          pltpu.VMEM((ROWS_PER_WORKER // 2, PAD_DIM), jnp.float32),
          pltpu.VMEM((ROWS_PER_WORKER,), jnp.float32),
          pltpu.SemaphoreType.DMA,
          pltpu.SemaphoreType.DMA,
          pltpu.SemaphoreType.DMA,
          pltpu.SemaphoreType.DMA,
          pltpu.SemaphoreType.DMA,
      ],
  )
  return run(diagp, idx, q)


def kernel(h, r, t, diag):
  return _dist_mult(h, r, t, diag)
